# baseline (speedup 1.0000x reference)
import functools

import jax
import jax.numpy as jnp
from jax import lax
from jax.experimental import pallas as pl
from jax.experimental.pallas import tpu as pltpu
from jax.experimental.pallas import tpu_sc as plsc

N = 20000
K = 16
CIN = 3
COUT = 64

QB = 128
NQB = (N + QB - 1) // QB
N_PAD = NQB * QB
NC_PAD = 20480
CT = 2048
NCT = NC_PAD // CT
G = 16
NG = NC_PAD // G
GPT = CT // G
BIGF = 3.0e38
BIGI = 2 ** 30


def _bf16_round_tc(v):
    u = lax.bitcast_convert_type(v, jnp.int32)
    u2 = u + (jnp.bitwise_and(lax.shift_right_logical(u, 16), 1) + 0x7FFF)
    return lax.bitcast_convert_type(
        jnp.bitwise_and(u2, jnp.int32(-65536)), jnp.float32)


def _screen_body(xqt_ref, xc_ref, gidx_ref, gm_ref):
    i = pl.program_id(0)
    j = pl.program_id(1)
    qt = xqt_ref[...]
    xc = xc_ref[...]
    c0 = xc[:, 0:1]
    c1 = xc[:, 1:2]
    c2 = xc[:, 2:3]
    sqc = (c0 * c0 + c1 * c1) + c2 * c2
    qb0 = _bf16_round_tc(qt[0:1, :])
    qb1 = _bf16_round_tc(qt[1:2, :])
    qb2 = _bf16_round_tc(qt[2:3, :])
    dot = (_bf16_round_tc(c0) * qb0 + _bf16_round_tc(c1) * qb1
           + _bf16_round_tc(c2) * qb2)
    score = sqc - 2.0 * dot

    gm_ref[pl.ds(j * GPT, GPT), :] = jnp.min(
        score.reshape(GPT, G, QB), axis=1)

    @pl.when(j == i // (CT // QB))
    def _():
        cand = j * CT + lax.broadcasted_iota(jnp.int32, (CT, QB), 0)
        qid = i * QB + lax.broadcasted_iota(jnp.int32, (CT, QB), 1)
        masked = jnp.where(cand == qid, BIGF, score)
        gm_ref[pl.ds(j * GPT, GPT), :] = jnp.min(
            masked.reshape(GPT, G, QB), axis=1)

    @pl.when(j == NCT - 1)
    def _():
        gm = gm_ref[...]
        b = lax.bitcast_convert_type(gm, jnp.int32)
        b = jnp.bitwise_xor(
            b, jnp.bitwise_and(lax.shift_right_arithmetic(b, 31),
                               jnp.int32(0x7FFFFFFF)))
        riota = lax.broadcasted_iota(jnp.int32, (NG, QB), 0)
        keys = jnp.bitwise_or(jnp.bitwise_and(b, jnp.int32(-2048)), riota)
        rows = []
        for _ in range(K):
            m = jnp.min(keys, axis=0, keepdims=True)
            rows.append(jnp.bitwise_and(m, 2047))
            keys = jnp.where(keys == m, jnp.int32(0x7FFFFFFF), keys)
        gidx_ref[...] = jnp.concatenate(rows, axis=0)


def _screen(xqt_pad, x_pad):
    return pl.pallas_call(
        _screen_body,
        grid=(NQB, NCT),
        in_specs=[
            pl.BlockSpec((CIN, QB), lambda i, j: (0, i)),
            pl.BlockSpec((CT, CIN), lambda i, j: (j, 0)),
        ],
        out_specs=[
            pl.BlockSpec((K, QB), lambda i, j: (0, i)),
            pl.BlockSpec((NG, QB), lambda i, j: (0, i)),
        ],
        out_shape=[
            jax.ShapeDtypeStruct((K, N_PAD), jnp.int32),
            jax.ShapeDtypeStruct((NG, N_PAD), jnp.float32),
        ],
    )(xqt_pad, x_pad)[0]


RB = 400


def _lin_body(x_ref, wa_ref, wc_ref, b_ref, a_ref, c_ref):
    q = x_ref[...]
    pad = jnp.zeros((RB, 128 - COUT), jnp.float32)
    avals = jnp.dot(q, wa_ref[...], preferred_element_type=jnp.float32) + b_ref[...]
    a_ref[...] = jnp.concatenate([avals, pad], axis=1)
    cvals = jnp.dot(q, wc_ref[...], preferred_element_type=jnp.float32)
    c_ref[...] = jnp.concatenate([cvals, pad], axis=1)


def _linear(x, wa, wc, b2):
    return pl.pallas_call(
        _lin_body,
        grid=(N // RB,),
        in_specs=[
            pl.BlockSpec((RB, CIN), lambda i: (i, 0)),
            pl.BlockSpec((CIN, COUT), lambda i: (0, 0)),
            pl.BlockSpec((CIN, COUT), lambda i: (0, 0)),
            pl.BlockSpec((1, COUT), lambda i: (0, 0)),
        ],
        out_specs=[
            pl.BlockSpec((RB, 128), lambda i: (i, 0)),
            pl.BlockSpec((RB, 128), lambda i: (i, 0)),
        ],
        out_shape=[
            jax.ShapeDtypeStruct((N, 128), jnp.float32),
            jax.ShapeDtypeStruct((N, 128), jnp.float32),
        ],
    )(x, wa, wc, b2)


_CH = 8
_NCHUNK = N // _CH


def _bf16_round(v):
    u = plsc.bitcast(v, jnp.int32)
    u2 = u + (jnp.bitwise_and(lax.shift_right_logical(u, 16), 1) + 0x7FFF)
    return plsc.bitcast(jnp.bitwise_and(u2, jnp.int32(-65536)), jnp.float32)


def _merge16(ak, av, bk, bv):
    bk_r = lax.rev(bk, (0,))
    bv_r = lax.rev(bv, (0,))
    take_a = ak <= bk_r
    mk = jnp.minimum(ak, bk_r)
    mv = jnp.where(take_a, av, bv_r)
    return plsc.sort_key_val(mk, mv)


_NXPAD = 20480


def _sc_refine(x0, x1, x2, gidx, c_tab, a_tab):
    info = plsc.get_sparse_core_info()
    nw = info.num_cores * info.num_subcores
    nchunk_per_w = (_NCHUNK + nw - 1) // nw

    mesh = plsc.VectorSubcoreMesh(core_axis_name="c", subcore_axis_name="s")

    @functools.partial(
        pl.kernel,
        out_type=jax.ShapeDtypeStruct((N, 128), jnp.float32),
        mesh=mesh,
        scratch_types=[
            pltpu.VMEM((_NXPAD,), jnp.float32),
            pltpu.VMEM((_NXPAD,), jnp.float32),
            pltpu.VMEM((_NXPAD,), jnp.float32),
            pltpu.VMEM((_CH * K,), jnp.int32),
            pltpu.VMEM((_CH * K,), jnp.int32),
            pltpu.VMEM((_CH * K, 128), jnp.float32),
            pltpu.VMEM((_CH, 128), jnp.float32),
            pltpu.VMEM((_CH, 128), jnp.float32),
            pltpu.SemaphoreType.DMA,
        ],
        compiler_params=pltpu.CompilerParams(needs_layout_passes=False),
    )
    def sc_body(x0_hbm, x1_hbm, x2_hbm, gidx_hbm, c_hbm, a_hbm, out_hbm,
                x0_v, x1_v, x2_v, g_v, cidx_v, rows_v, a_v, out_v, sem):
        wid = lax.axis_index("s") * info.num_cores + lax.axis_index("c")
        pltpu.sync_copy(x0_hbm, x0_v)
        pltpu.sync_copy(x1_hbm, x1_v)
        pltpu.sync_copy(x2_hbm, x2_v)
        lane = lax.broadcasted_iota(jnp.int32, (K,), 0)

        def chunk_body(t, _):
            c = t * nw + wid

            @pl.when(c < _NCHUNK)
            def _():
                base = c * _CH
                pltpu.sync_copy(gidx_hbm.at[pl.ds(base * K, _CH * K)], g_v)
                pltpu.sync_copy(a_hbm.at[pl.ds(base, _CH)], a_v)
                for n in range(_CH):
                    r = base + n
                    rsplat = jnp.full((K,), r, jnp.int32)
                    q0 = _bf16_round(plsc.load_gather(x0_v, [rsplat]))
                    q1 = _bf16_round(plsc.load_gather(x1_v, [rsplat]))
                    q2 = _bf16_round(plsc.load_gather(x2_v, [rsplat]))
                    g = plsc.load_gather(g_v, [n * K + lane])
                    lists = []
                    for j in range(G):
                        cj = g * G + j
                        c0 = plsc.load_gather(x0_v, [cj])
                        c1 = plsc.load_gather(x1_v, [cj])
                        c2 = plsc.load_gather(x2_v, [cj])
                        sqc = c0 * c0 + c1 * c1 + c2 * c2
                        dot = (_bf16_round(c0) * q0 + _bf16_round(c1) * q1
                               + _bf16_round(c2) * q2)
                        s = sqc - 2.0 * dot
                        s = jnp.where(cj == rsplat, BIGF, s)
                        lists.append(plsc.sort_key_val(s, cj))
                    while len(lists) > 1:
                        nxt = []
                        for p in range(0, len(lists), 2):
                            ak, av = lists[p]
                            bk, bv = lists[p + 1]
                            nxt.append(_merge16(ak, av, bk, bv))
                        lists = nxt
                    cidx_v[pl.ds(n * K, K)] = lists[0][1]
                pltpu.async_copy(c_hbm.at[cidx_v], rows_v, sem).wait()
                for n in range(_CH):
                    for gg in range(COUT // 16):
                        sl = pl.ds(gg * 16, 16)
                        acc = rows_v[n * K, sl]
                        for rr in range(1, K):
                            acc = jnp.maximum(acc, rows_v[n * K + rr, sl])
                        out_v[n, sl] = jnp.maximum(acc + a_v[n, sl], 0.0)
                pltpu.sync_copy(out_v, out_hbm.at[pl.ds(base, _CH)])
            return _

        lax.fori_loop(0, nchunk_per_w, chunk_body, None)

    return sc_body(x0, x1, x2, gidx, c_tab, a_tab)


def kernel(x, W, b):
    xt = x.T
    xqt_pad = jnp.concatenate(
        [xt, jnp.full((CIN, N_PAD - N), 1.0e9, jnp.float32)], axis=1)
    wa = (W[:, :CIN] - W[:, CIN:]).T
    wc = W[:, CIN:].T
    b2 = b.reshape(1, COUT)
    x_pad = jnp.concatenate(
        [x, jnp.full((NC_PAD - N, CIN), 1.0e9, jnp.float32)], axis=0)
    gidx = _screen(xqt_pad, x_pad)[:, :N].T.reshape(-1)
    a_tab, c_tab = _linear(x, wa, wc, b2)
    zpad = jnp.full((_NXPAD - N,), 1.0e9, jnp.float32)
    x0 = jnp.concatenate([xt[0], zpad])
    x1 = jnp.concatenate([xt[1], zpad])
    x2 = jnp.concatenate([xt[2], zpad])
    out = _sc_refine(x0, x1, x2, gidx, c_tab, a_tab)
    return out[:, :COUT]


# speedup vs baseline: 1.1918x; 1.1918x over previous
import functools

import jax
import jax.numpy as jnp
from jax import lax
from jax.experimental import pallas as pl
from jax.experimental.pallas import tpu as pltpu
from jax.experimental.pallas import tpu_sc as plsc

N = 20000
K = 16
CIN = 3
COUT = 64

QB = 128
NQB = (N + QB - 1) // QB
N_PAD = NQB * QB
NC_PAD = 20480
CT = 2048
NCT = NC_PAD // CT
G = 16
NG = NC_PAD // G
GPT = CT // G
BIGF = 3.0e38
BIGI = 2 ** 30


def _bf16_round_tc(v):
    u = lax.bitcast_convert_type(v, jnp.int32)
    u2 = u + (jnp.bitwise_and(lax.shift_right_logical(u, 16), 1) + 0x7FFF)
    return lax.bitcast_convert_type(
        jnp.bitwise_and(u2, jnp.int32(-65536)), jnp.float32)


def _screen_body(xqt_ref, xc_ref, gidx_ref, gm_ref):
    i = pl.program_id(0)
    j = pl.program_id(1)
    qt = xqt_ref[...]
    xc = xc_ref[...]
    c0 = xc[:, 0:1]
    c1 = xc[:, 1:2]
    c2 = xc[:, 2:3]
    sqc = (c0 * c0 + c1 * c1) + c2 * c2
    score = sqc - 2.0 * jnp.dot(xc, qt, preferred_element_type=jnp.float32)

    gm_ref[pl.ds(j * GPT, GPT), :] = jnp.min(
        score.reshape(GPT, G, QB), axis=1)

    @pl.when(j == i // (CT // QB))
    def _():
        cand = j * CT + lax.broadcasted_iota(jnp.int32, (CT, QB), 0)
        qid = i * QB + lax.broadcasted_iota(jnp.int32, (CT, QB), 1)
        masked = jnp.where(cand == qid, BIGF, score)
        gm_ref[pl.ds(j * GPT, GPT), :] = jnp.min(
            masked.reshape(GPT, G, QB), axis=1)

    @pl.when(j == NCT - 1)
    def _():
        gm = gm_ref[...]
        b = lax.bitcast_convert_type(gm, jnp.int32)
        b = jnp.bitwise_xor(
            b, jnp.bitwise_and(lax.shift_right_arithmetic(b, 31),
                               jnp.int32(0x7FFFFFFF)))
        riota = lax.broadcasted_iota(jnp.int32, (NG, QB), 0)
        keys = jnp.bitwise_or(jnp.bitwise_and(b, jnp.int32(-2048)), riota)
        rows = []
        for _ in range(K):
            m = jnp.min(keys, axis=0, keepdims=True)
            rows.append(jnp.bitwise_and(m, 2047))
            keys = jnp.where(keys == m, jnp.int32(0x7FFFFFFF), keys)
        gidx_ref[...] = jnp.concatenate(rows, axis=0)


def _screen(xqt_pad, x_pad):
    return pl.pallas_call(
        _screen_body,
        grid=(NQB, NCT),
        in_specs=[
            pl.BlockSpec((CIN, QB), lambda i, j: (0, i)),
            pl.BlockSpec((CT, CIN), lambda i, j: (j, 0)),
        ],
        out_specs=[
            pl.BlockSpec((K, QB), lambda i, j: (0, i)),
            pl.BlockSpec((NG, QB), lambda i, j: (0, i)),
        ],
        out_shape=[
            jax.ShapeDtypeStruct((K, N_PAD), jnp.int32),
            jax.ShapeDtypeStruct((NG, N_PAD), jnp.float32),
        ],
    )(xqt_pad, x_pad)[0]


RB = 400


def _lin_body(x_ref, wa_ref, wc_ref, b_ref, a_ref, c_ref):
    q = x_ref[...]
    pad = jnp.zeros((RB, 128 - COUT), jnp.float32)
    avals = jnp.dot(q, wa_ref[...], preferred_element_type=jnp.float32) + b_ref[...]
    a_ref[...] = jnp.concatenate([avals, pad], axis=1)
    cvals = jnp.dot(q, wc_ref[...], preferred_element_type=jnp.float32)
    c_ref[...] = jnp.concatenate([cvals, pad], axis=1)


def _linear(x, wa, wc, b2):
    return pl.pallas_call(
        _lin_body,
        grid=(N // RB,),
        in_specs=[
            pl.BlockSpec((RB, CIN), lambda i: (i, 0)),
            pl.BlockSpec((CIN, COUT), lambda i: (0, 0)),
            pl.BlockSpec((CIN, COUT), lambda i: (0, 0)),
            pl.BlockSpec((1, COUT), lambda i: (0, 0)),
        ],
        out_specs=[
            pl.BlockSpec((RB, 128), lambda i: (i, 0)),
            pl.BlockSpec((RB, 128), lambda i: (i, 0)),
        ],
        out_shape=[
            jax.ShapeDtypeStruct((N, 128), jnp.float32),
            jax.ShapeDtypeStruct((N, 128), jnp.float32),
        ],
    )(x, wa, wc, b2)


_CH = 8
_NCHUNK = N // _CH


def _bf16_round(v):
    u = plsc.bitcast(v, jnp.int32)
    u2 = u + (jnp.bitwise_and(lax.shift_right_logical(u, 16), 1) + 0x7FFF)
    return plsc.bitcast(jnp.bitwise_and(u2, jnp.int32(-65536)), jnp.float32)


def _merge16(ak, av, bk, bv):
    bk_r = lax.rev(bk, (0,))
    bv_r = lax.rev(bv, (0,))
    take_a = ak <= bk_r
    mk = jnp.minimum(ak, bk_r)
    mv = jnp.where(take_a, av, bv_r)
    return plsc.sort_key_val(mk, mv)


_NXPAD = 20480


def _sc_refine(x0, x1, x2, gidx, c_tab, a_tab):
    info = plsc.get_sparse_core_info()
    nw = info.num_cores * info.num_subcores
    nchunk_per_w = (_NCHUNK + nw - 1) // nw

    mesh = plsc.VectorSubcoreMesh(core_axis_name="c", subcore_axis_name="s")

    @functools.partial(
        pl.kernel,
        out_type=jax.ShapeDtypeStruct((N, 128), jnp.float32),
        mesh=mesh,
        scratch_types=[
            pltpu.VMEM((_NXPAD,), jnp.float32),
            pltpu.VMEM((_NXPAD,), jnp.float32),
            pltpu.VMEM((_NXPAD,), jnp.float32),
            pltpu.VMEM((_CH * K,), jnp.int32),
            pltpu.VMEM((_CH * K,), jnp.int32),
            pltpu.VMEM((_CH * K, 128), jnp.float32),
            pltpu.VMEM((_CH, 128), jnp.float32),
            pltpu.VMEM((_CH, 128), jnp.float32),
            pltpu.SemaphoreType.DMA,
        ],
        compiler_params=pltpu.CompilerParams(needs_layout_passes=False),
    )
    def sc_body(x0_hbm, x1_hbm, x2_hbm, gidx_hbm, c_hbm, a_hbm, out_hbm,
                x0_v, x1_v, x2_v, g_v, cidx_v, rows_v, a_v, out_v, sem):
        wid = lax.axis_index("s") * info.num_cores + lax.axis_index("c")
        pltpu.sync_copy(x0_hbm, x0_v)
        pltpu.sync_copy(x1_hbm, x1_v)
        pltpu.sync_copy(x2_hbm, x2_v)
        lane = lax.broadcasted_iota(jnp.int32, (K,), 0)

        def chunk_body(t, _):
            c = t * nw + wid

            @pl.when(c < _NCHUNK)
            def _():
                base = c * _CH
                pltpu.sync_copy(gidx_hbm.at[pl.ds(base * K, _CH * K)], g_v)
                pltpu.sync_copy(a_hbm.at[pl.ds(base, _CH)], a_v)
                for n in range(_CH):
                    r = base + n
                    rsplat = jnp.full((K,), r, jnp.int32)
                    q0 = _bf16_round(plsc.load_gather(x0_v, [rsplat]))
                    q1 = _bf16_round(plsc.load_gather(x1_v, [rsplat]))
                    q2 = _bf16_round(plsc.load_gather(x2_v, [rsplat]))
                    g = plsc.load_gather(g_v, [n * K + lane])
                    lists = []
                    for j in range(G):
                        cj = g * G + j
                        c0 = plsc.load_gather(x0_v, [cj])
                        c1 = plsc.load_gather(x1_v, [cj])
                        c2 = plsc.load_gather(x2_v, [cj])
                        sqc = c0 * c0 + c1 * c1 + c2 * c2
                        dot = (_bf16_round(c0) * q0 + _bf16_round(c1) * q1
                               + _bf16_round(c2) * q2)
                        s = sqc - 2.0 * dot
                        s = jnp.where(cj == rsplat, BIGF, s)
                        lists.append(plsc.sort_key_val(s, cj))
                    while len(lists) > 1:
                        nxt = []
                        for p in range(0, len(lists), 2):
                            ak, av = lists[p]
                            bk, bv = lists[p + 1]
                            nxt.append(_merge16(ak, av, bk, bv))
                        lists = nxt
                    cidx_v[pl.ds(n * K, K)] = lists[0][1]
                pltpu.async_copy(c_hbm.at[cidx_v], rows_v, sem).wait()
                for n in range(_CH):
                    for gg in range(COUT // 16):
                        sl = pl.ds(gg * 16, 16)
                        acc = rows_v[n * K, sl]
                        for rr in range(1, K):
                            acc = jnp.maximum(acc, rows_v[n * K + rr, sl])
                        out_v[n, sl] = jnp.maximum(acc + a_v[n, sl], 0.0)
                pltpu.sync_copy(out_v, out_hbm.at[pl.ds(base, _CH)])
            return _

        lax.fori_loop(0, nchunk_per_w, chunk_body, None)

    return sc_body(x0, x1, x2, gidx, c_tab, a_tab)


def kernel(x, W, b):
    xt = x.T
    xqt_pad = jnp.concatenate(
        [xt, jnp.full((CIN, N_PAD - N), 1.0e9, jnp.float32)], axis=1)
    wa = (W[:, :CIN] - W[:, CIN:]).T
    wc = W[:, CIN:].T
    b2 = b.reshape(1, COUT)
    x_pad = jnp.concatenate(
        [x, jnp.full((NC_PAD - N, CIN), 1.0e9, jnp.float32)], axis=0)
    gidx = _screen(xqt_pad, x_pad)[:, :N].T.reshape(-1)
    a_tab, c_tab = _linear(x, wa, wc, b2)
    zpad = jnp.full((_NXPAD - N,), 1.0e9, jnp.float32)
    x0 = jnp.concatenate([xt[0], zpad])
    x1 = jnp.concatenate([xt[1], zpad])
    x2 = jnp.concatenate([xt[2], zpad])
    out = _sc_refine(x0, x1, x2, gidx, c_tab, a_tab)
    return out[:, :COUT]
